# baseline (speedup 1.0000x reference)
import functools

import jax
import jax.numpy as jnp
from jax import lax
from jax.experimental import pallas as pl
from jax.experimental.pallas import tpu as pltpu
from jax.experimental.pallas import tpu_sc as plsc

B = 4096
L = 128
V = 100000
D = 128
H = 1024
O = 256

_V_PAD = 102400
_VHALF = _V_PAD // 2
_QW = _VHALF // 2
_Q_BLK = 5120


def _pack_bf16(u):
    return (u + 0x7FFF + ((u >> 16) & 1)) >> 16


def _rowmean_body(wlo_ref, whi_ref, rt_ref):
    ones_row = jnp.full((1, D), 1.0 / D, dtype=jnp.float32)
    dims = (((1,), (1,)), ((), ()))
    s_lo = jax.lax.dot_general(ones_row, wlo_ref[...], dims,
                               preferred_element_type=jnp.float32)
    s_hi = jax.lax.dot_general(ones_row, whi_ref[...], dims,
                               preferred_element_type=jnp.float32)
    b_lo = _pack_bf16(jax.lax.bitcast_convert_type(s_lo, jnp.int32))
    b_hi = _pack_bf16(jax.lax.bitcast_convert_type(s_hi, jnp.int32))
    word = (b_lo & 0xFFFF) | (b_hi << 16)
    rt_ref[...] = word.reshape(_Q_BLK)


def _rowmean_packed_half(weights, half):
    nblk = _QW // _Q_BLK
    lo0 = half * _VHALF // _Q_BLK
    hi0 = lo0 + nblk
    return pl.pallas_call(
        _rowmean_body,
        grid=(nblk,),
        in_specs=[
            pl.BlockSpec((_Q_BLK, D), lambda i, r=lo0: (i + r, 0)),
            pl.BlockSpec((_Q_BLK, D), lambda i, r=hi0: (i + r, 0)),
        ],
        out_specs=pl.BlockSpec((_Q_BLK,), lambda i: (i,)),
        out_shape=jax.ShapeDtypeStruct((_QW,), jnp.int32),
    )(weights, weights)


_BL = B * L
_CHUNK = 8192


def _sc_gather(rt, idx, half):
    mesh = plsc.VectorSubcoreMesh(core_axis_name="c", subcore_axis_name="s",
                                  num_subcores=8)
    nw = mesh.num_cores * mesh.num_subcores
    per_w = _BL // nw
    nchunk = per_w // _CHUNK
    base_off = half * _VHALF

    @functools.partial(
        pl.kernel,
        out_type=jax.ShapeDtypeStruct((_BL,), jnp.float32),
        mesh=mesh,
        scratch_types=[
            pltpu.VMEM((_QW,), jnp.int32),
            pltpu.VMEM((_CHUNK,), jnp.int32),
            pltpu.VMEM((_CHUNK,), jnp.int32),
            pltpu.VMEM((_CHUNK,), jnp.float32),
            pltpu.VMEM((_CHUNK,), jnp.float32),
            pltpu.SemaphoreType.DMA,
            pltpu.SemaphoreType.DMA,
            pltpu.SemaphoreType.DMA,
            pltpu.SemaphoreType.DMA,
            pltpu.SemaphoreType.DMA,
        ],
        compiler_params=pltpu.CompilerParams(needs_layout_passes=False),
    )
    def gather_kernel(rt_hbm, idx_hbm, out_hbm, rt_v, idx_a, idx_b,
                      val_a, val_b, sem_r, sem_ia, sem_ib, sem_oa, sem_ob):
        wid = lax.axis_index("s") * mesh.num_cores + lax.axis_index("c")
        base = wid * per_w
        idx_bufs = (idx_a, idx_b)
        val_bufs = (val_a, val_b)
        idx_sems = (sem_ia, sem_ib)
        out_sems = (sem_oa, sem_ob)

        r_cp = pltpu.async_copy(rt_hbm, rt_v, sem_r)
        idx_cps = [None, None]
        out_cps = [None, None]
        idx_cps[0] = pltpu.async_copy(
            idx_hbm.at[pl.ds(base, _CHUNK)], idx_bufs[0], idx_sems[0])
        r_cp.wait()
        for c in range(nchunk):
            p = c % 2
            q = (c + 1) % 2
            if c + 1 < nchunk:
                idx_cps[q] = pltpu.async_copy(
                    idx_hbm.at[pl.ds(base + (c + 1) * _CHUNK, _CHUNK)],
                    idx_bufs[q], idx_sems[q])
            idx_cps[p].wait()
            if out_cps[p] is not None:
                out_cps[p].wait()
            idx_v = idx_bufs[p]
            val_v = val_bufs[p]

            def body(j, carry):
                grp = j * 128
                idxs = [idx_v[pl.ds(grp + k * 16, 16)] for k in range(8)]
                vals = []
                for ix in idxs:
                    l16 = ix - base_off
                    l16 = jnp.minimum(jnp.maximum(l16, 0), _VHALF - 1)
                    hi = l16 >= _QW
                    k16 = l16 - jnp.where(hi, _QW, 0)
                    w = plsc.load_gather(rt_v, [k16])
                    bits = jnp.where(hi, w & jnp.int32(-65536), w << 16)
                    vals.append(plsc.bitcast(bits, jnp.float32))
                for k in range(8):
                    val_v[pl.ds(grp + k * 16, 16)] = vals[k]
                return carry

            lax.fori_loop(0, _CHUNK // 128, body, 0)
            out_cps[p] = pltpu.async_copy(
                val_v, out_hbm.at[pl.ds(base + c * _CHUNK, _CHUNK)], out_sems[p])
        for cp in out_cps:
            if cp is not None:
                cp.wait()

    return gather_kernel(rt, idx)


_B_BLK = 1024


def _mlp_body(x_ref, ma_ref, mb_ref, w1_ref, b1_ref, w2_ref, b2_ref,
              o_ref, wc_ref, bc_ref):
    i = pl.program_id(0)

    @pl.when(i == 0)
    def _():
        wc_ref[...] = jnp.dot(w1_ref[...], w2_ref[...],
                              preferred_element_type=jnp.float32)
        bc_ref[...] = jnp.dot(b1_ref[...], w2_ref[...],
                              preferred_element_type=jnp.float32) + b2_ref[...]

    m = jnp.where(x_ref[...] < _VHALF, ma_ref[...], mb_ref[...])
    z = jnp.dot(m, wc_ref[...],
                preferred_element_type=jnp.float32) + bc_ref[...]
    z = z - jnp.max(z, axis=1, keepdims=True)
    e = jnp.exp(z)
    o_ref[...] = e / jnp.sum(e, axis=1, keepdims=True)


def _mlp_softmax(x2d, ma, mb, W1, b1, W2, b2):
    return pl.pallas_call(
        _mlp_body,
        grid=(B // _B_BLK,),
        in_specs=[
            pl.BlockSpec((_B_BLK, L), lambda i: (i, 0)),
            pl.BlockSpec((_B_BLK, L), lambda i: (i, 0)),
            pl.BlockSpec((_B_BLK, L), lambda i: (i, 0)),
            pl.BlockSpec((D, H), lambda i: (0, 0)),
            pl.BlockSpec((1, H), lambda i: (0, 0)),
            pl.BlockSpec((H, O), lambda i: (0, 0)),
            pl.BlockSpec((1, O), lambda i: (0, 0)),
        ],
        out_specs=pl.BlockSpec((_B_BLK, O), lambda i: (i, 0)),
        out_shape=jax.ShapeDtypeStruct((B, O), jnp.float32),
        scratch_shapes=[
            pltpu.VMEM((L, O), jnp.float32),
            pltpu.VMEM((1, O), jnp.float32),
        ],
    )(x2d, ma, mb, W1, b1.reshape(1, H), W2, b2.reshape(1, O))


def kernel(x, weights, W1, b1, W2, b2):
    xi = x.astype(jnp.int32)
    idx = xi.reshape(_BL)
    rt0 = _rowmean_packed_half(weights, 0)
    m0 = _sc_gather(rt0, idx, 0)
    rt1 = _rowmean_packed_half(weights, 1)
    m1 = _sc_gather(rt1, idx, 1)
    return _mlp_softmax(xi, m0.reshape(B, L), m1.reshape(B, L),
                        W1, b1, W2, b2)


# speedup vs baseline: 1.2501x; 1.2501x over previous
import functools

import jax
import jax.numpy as jnp
from jax import lax
from jax.experimental import pallas as pl
from jax.experimental.pallas import tpu as pltpu
from jax.experimental.pallas import tpu_sc as plsc

B = 4096
L = 128
V = 100000
D = 128
H = 1024
O = 256

_V_PAD = 102400
_VHALF = _V_PAD // 2
_HALF_BLK = 10240


def _pack_bf16(u):
    return (u + 0x7FFF + ((u >> 16) & 1)) >> 16


def _rowmean_body(wlo_ref, whi_ref, rt_ref):
    ones_row = jnp.full((1, D), 1.0 / D, dtype=jnp.float32)
    dims = (((1,), (1,)), ((), ()))
    s_lo = jax.lax.dot_general(ones_row, wlo_ref[...], dims,
                               preferred_element_type=jnp.float32)
    s_hi = jax.lax.dot_general(ones_row, whi_ref[...], dims,
                               preferred_element_type=jnp.float32)
    b_lo = _pack_bf16(jax.lax.bitcast_convert_type(s_lo, jnp.int32))
    b_hi = _pack_bf16(jax.lax.bitcast_convert_type(s_hi, jnp.int32))
    word = (b_lo & 0xFFFF) | (b_hi << 16)
    rt_ref[...] = word.reshape(_HALF_BLK)


def _rowmean_packed(weights):
    nblk = _VHALF // _HALF_BLK
    return pl.pallas_call(
        _rowmean_body,
        grid=(nblk,),
        in_specs=[
            pl.BlockSpec((_HALF_BLK, D), lambda i: (i, 0)),
            pl.BlockSpec((_HALF_BLK, D), lambda i, n=nblk: (i + n, 0)),
        ],
        out_specs=pl.BlockSpec((_HALF_BLK,), lambda i: (i,)),
        out_shape=jax.ShapeDtypeStruct((_VHALF,), jnp.int32),
    )(weights, weights)


_BL = B * L
_CHUNK = 8192


def _sc_gather(rt, idx):
    mesh = plsc.VectorSubcoreMesh(core_axis_name="c", subcore_axis_name="s",
                                  num_subcores=8)
    nw = mesh.num_cores * mesh.num_subcores
    per_w = _BL // nw
    nchunk = per_w // _CHUNK

    @functools.partial(
        pl.kernel,
        out_type=jax.ShapeDtypeStruct((_BL,), jnp.float32),
        mesh=mesh,
        scratch_types=[
            pltpu.VMEM((_VHALF,), jnp.int32),
            pltpu.VMEM((_CHUNK,), jnp.int32),
            pltpu.VMEM((_CHUNK,), jnp.int32),
            pltpu.VMEM((_CHUNK,), jnp.float32),
            pltpu.VMEM((_CHUNK,), jnp.float32),
            pltpu.SemaphoreType.DMA,
            pltpu.SemaphoreType.DMA,
            pltpu.SemaphoreType.DMA,
            pltpu.SemaphoreType.DMA,
            pltpu.SemaphoreType.DMA,
        ],
        compiler_params=pltpu.CompilerParams(needs_layout_passes=False),
    )
    def gather_kernel(rt_hbm, idx_hbm, out_hbm, rt_v, idx_a, idx_b,
                      val_a, val_b, sem_r, sem_ia, sem_ib, sem_oa, sem_ob):
        wid = lax.axis_index("s") * mesh.num_cores + lax.axis_index("c")
        base = wid * per_w
        idx_bufs = (idx_a, idx_b)
        val_bufs = (val_a, val_b)
        idx_sems = (sem_ia, sem_ib)
        out_sems = (sem_oa, sem_ob)

        r_cp = pltpu.async_copy(rt_hbm, rt_v, sem_r)
        idx_cps = [None, None]
        out_cps = [None, None]
        idx_cps[0] = pltpu.async_copy(
            idx_hbm.at[pl.ds(base, _CHUNK)], idx_bufs[0], idx_sems[0])
        r_cp.wait()
        for c in range(nchunk):
            p = c % 2
            q = (c + 1) % 2
            if c + 1 < nchunk:
                idx_cps[q] = pltpu.async_copy(
                    idx_hbm.at[pl.ds(base + (c + 1) * _CHUNK, _CHUNK)],
                    idx_bufs[q], idx_sems[q])
            idx_cps[p].wait()
            if out_cps[p] is not None:
                out_cps[p].wait()
            idx_v = idx_bufs[p]
            val_v = val_bufs[p]

            def body(j, carry):
                grp = j * 128
                idxs = [idx_v[pl.ds(grp + k * 16, 16)] for k in range(8)]
                vals = []
                for ix in idxs:
                    hi = ix >= _VHALF
                    k16 = ix - jnp.where(hi, _VHALF, 0)
                    w = plsc.load_gather(rt_v, [k16])
                    bits = jnp.where(hi, w & jnp.int32(-65536), w << 16)
                    vals.append(plsc.bitcast(bits, jnp.float32))
                for k in range(8):
                    val_v[pl.ds(grp + k * 16, 16)] = vals[k]
                return carry

            lax.fori_loop(0, _CHUNK // 128, body, 0)
            out_cps[p] = pltpu.async_copy(
                val_v, out_hbm.at[pl.ds(base + c * _CHUNK, _CHUNK)], out_sems[p])
        for cp in out_cps:
            if cp is not None:
                cp.wait()

    return gather_kernel(rt, idx)


_B_BLK = 1024


def _mlp_body(m_ref, w1_ref, b1_ref, w2_ref, b2_ref, o_ref, wc_ref, bc_ref):
    i = pl.program_id(0)

    @pl.when(i == 0)
    def _():
        wc_ref[...] = jnp.dot(w1_ref[...], w2_ref[...],
                              preferred_element_type=jnp.float32)
        bc_ref[...] = jnp.dot(b1_ref[...], w2_ref[...],
                              preferred_element_type=jnp.float32) + b2_ref[...]

    z = jnp.dot(m_ref[...], wc_ref[...],
                preferred_element_type=jnp.float32) + bc_ref[...]
    z = z - jnp.max(z, axis=1, keepdims=True)
    e = jnp.exp(z)
    o_ref[...] = e / jnp.sum(e, axis=1, keepdims=True)


def _mlp_softmax(m, W1, b1, W2, b2):
    return pl.pallas_call(
        _mlp_body,
        grid=(B // _B_BLK,),
        in_specs=[
            pl.BlockSpec((_B_BLK, L), lambda i: (i, 0)),
            pl.BlockSpec((D, H), lambda i: (0, 0)),
            pl.BlockSpec((1, H), lambda i: (0, 0)),
            pl.BlockSpec((H, O), lambda i: (0, 0)),
            pl.BlockSpec((1, O), lambda i: (0, 0)),
        ],
        out_specs=pl.BlockSpec((_B_BLK, O), lambda i: (i, 0)),
        out_shape=jax.ShapeDtypeStruct((B, O), jnp.float32),
        scratch_shapes=[
            pltpu.VMEM((L, O), jnp.float32),
            pltpu.VMEM((1, O), jnp.float32),
        ],
    )(m, W1, b1.reshape(1, H), W2, b2.reshape(1, O))


def kernel(x, weights, W1, b1, W2, b2):
    rt = _rowmean_packed(weights)
    idx = x.reshape(_BL).astype(jnp.int32)
    m = _sc_gather(rt, idx)
    return _mlp_softmax(m.reshape(B, L), W1, b1, W2, b2)
